# initial kernel scaffold (unmeasured)
import jax
import jax.numpy as jnp
from jax import lax
from jax.experimental import pallas as pl
from jax.experimental.pallas import tpu as pltpu

N_DEV = 4
SQ = 1024
SKV = 1024
D_MODEL = 1024
H_PER = 8
DH = 128
SCALE = 0.08838834764831843
NEG = -1e9


def _body(x_ref, wqo_ref, k_ref, v_ref, out_ref, comm_ref, send_sems, recv_sems):
    my = lax.axis_index("i")
    right = lax.rem(my + 1, N_DEV)
    left = lax.rem(my + N_DEV - 1, N_DEV)

    barrier_sem = pltpu.get_barrier_semaphore()
    for nbr in (left, right):
        pl.semaphore_signal(
            barrier_sem, inc=1,
            device_id=(nbr,), device_id_type=pl.DeviceIdType.MESH,
        )
    pl.semaphore_wait(barrier_sem, 2)

    x = x_ref[...]

    qb = lax.broadcasted_iota(jnp.int32, (SQ, SKV), 0) // 64
    kb = lax.broadcasted_iota(jnp.int32, (SQ, SKV), 1) // 64
    mask = (qb == kb) | (kb == 0) | (lax.rem(qb + kb, 3) == 0)

    def compute_group(g, wq_g, wo_g):
        qg = jnp.dot(x, wq_g, preferred_element_type=jnp.float32)
        kg = k_ref[pl.ds(g * H_PER, H_PER), :, :]
        vg = v_ref[pl.ds(g * H_PER, H_PER), :, :]
        ctxs = []
        for h in range(H_PER):
            qh = qg[:, h * DH:(h + 1) * DH]
            s = lax.dot_general(
                qh, kg[h],
                (((1,), (1,)), ((), ())),
                preferred_element_type=jnp.float32,
            ) * SCALE
            s = jnp.where(mask, s, NEG)
            w = jnp.exp(s - jnp.max(s, axis=1, keepdims=True))
            w = w / jnp.sum(w, axis=1, keepdims=True)
            ctxs.append(jnp.dot(w, vg[h], preferred_element_type=jnp.float32))
        ctx = jnp.concatenate(ctxs, axis=1)
        return jnp.dot(ctx, wo_g, preferred_element_type=jnp.float32)

    out_ref[...] = compute_group(my, wqo_ref[0], wqo_ref[1])

    for h in range(N_DEV - 1):
        src = wqo_ref if h == 0 else comm_ref.at[h - 1]
        rdma = pltpu.make_async_remote_copy(
            src_ref=src,
            dst_ref=comm_ref.at[h],
            send_sem=send_sems.at[h],
            recv_sem=recv_sems.at[h],
            device_id=(right,),
            device_id_type=pl.DeviceIdType.MESH,
        )
        rdma.start()
        rdma.wait()
        g = lax.rem(my + N_DEV - 1 - h, N_DEV)
        out_ref[...] = out_ref[...] + compute_group(
            g, comm_ref[h, 0], comm_ref[h, 1]
        )


def kernel(x, Wq, K_ext, V_ext, Wo):
    my = lax.axis_index("i")
    xb = x[0]
    kb = jnp.swapaxes(lax.dynamic_index_in_dim(K_ext, my, 0, keepdims=False), 0, 1)
    vb = jnp.swapaxes(lax.dynamic_index_in_dim(V_ext, my, 0, keepdims=False), 0, 1)
    wqo = jnp.stack([Wq, Wo])

    out = pl.pallas_call(
        _body,
        out_shape=jax.ShapeDtypeStruct((SQ, D_MODEL), jnp.float32),
        in_specs=[
            pl.BlockSpec(memory_space=pltpu.VMEM),
            pl.BlockSpec(memory_space=pltpu.VMEM),
            pl.BlockSpec(memory_space=pltpu.VMEM),
            pl.BlockSpec(memory_space=pltpu.VMEM),
        ],
        out_specs=pl.BlockSpec(memory_space=pltpu.VMEM),
        scratch_shapes=[
            pltpu.VMEM((N_DEV - 1, 2, D_MODEL, D_MODEL), jnp.float32),
            pltpu.SemaphoreType.DMA((N_DEV - 1,)),
            pltpu.SemaphoreType.DMA((N_DEV - 1,)),
        ],
        compiler_params=pltpu.CompilerParams(collective_id=0),
    )(xb, wqo, kb, vb)
    return out[None]


# baseline (device time: 519315 ns/iter reference)
import jax
import jax.numpy as jnp
from jax import lax
from jax.experimental import pallas as pl
from jax.experimental.pallas import tpu as pltpu

N_DEV = 4
SQ = 1024
SKV = 1024
D_MODEL = 1024
H_PER = 8
DH = 128
SCALE = 0.08838834764831843


def _body(x_ref, wq_hbm, wo_hbm, k_hbm, v_hbm, out_ref, commq, commo,
          wq_st, wo_st, k_st, v_st, bias_ref,
          sendq, recvq, sendo, recvo, st_sems):
    my = lax.axis_index("i")
    right = lax.rem(my + 1, N_DEV)
    left = lax.rem(my + N_DEV - 1, N_DEV)

    barrier_sem = pltpu.get_barrier_semaphore()
    for nbr in (left, right):
        pl.semaphore_signal(
            barrier_sem, inc=1,
            device_id=(nbr,), device_id_type=pl.DeviceIdType.MESH,
        )
    pl.semaphore_wait(barrier_sem, 2)

    qb = lax.broadcasted_iota(jnp.int32, (SQ, SKV), 0) // 64
    kb = lax.broadcasted_iota(jnp.int32, (SQ, SKV), 1) // 64
    keep = (qb == kb) | (kb == 0) | (lax.rem(qb + kb, 3) == 0)
    bias_ref[...] = jnp.where(keep, jnp.float32(0.0), jnp.float32(-1e9))

    def stage_group(g, wq_src, wo_src):
        cps = [
            pltpu.make_async_copy(wq_src, wq_st, st_sems.at[0]),
            pltpu.make_async_copy(wo_src, wo_st, st_sems.at[1]),
            pltpu.make_async_copy(
                k_hbm.at[pl.ds(g * H_PER, H_PER)], k_st, st_sems.at[2]),
            pltpu.make_async_copy(
                v_hbm.at[pl.ds(g * H_PER, H_PER)], v_st, st_sems.at[3]),
        ]
        for cp in cps:
            cp.start()
        for cp in cps:
            cp.wait()

    def compute_group():
        def head_body(h, carry):
            qh = jnp.dot(x_ref[...], wq_st[h],
                         preferred_element_type=jnp.float32)
            s = lax.dot_general(
                qh, k_st[h],
                (((1,), (1,)), ((), ())),
                preferred_element_type=jnp.float32,
            ) * SCALE + bias_ref[...]
            w = jnp.exp(s - jnp.max(s, axis=1, keepdims=True))
            w = w / jnp.sum(w, axis=1, keepdims=True)
            ctx = jnp.dot(w, v_st[h],
                          preferred_element_type=jnp.float32)
            out_ref[...] = out_ref[...] + jnp.dot(
                ctx, wo_st[h], preferred_element_type=jnp.float32)
            return carry
        lax.fori_loop(0, H_PER, head_body, 0)

    out_ref[...] = jnp.zeros((SQ, D_MODEL), jnp.float32)

    stage_group(my, wq_hbm, wo_hbm)
    compute_group()

    for h in range(N_DEV - 1):
        srcq = wq_hbm if h == 0 else commq.at[h - 1]
        srco = wo_hbm if h == 0 else commo.at[h - 1]
        rq = pltpu.make_async_remote_copy(
            src_ref=srcq, dst_ref=commq.at[h],
            send_sem=sendq.at[h], recv_sem=recvq.at[h],
            device_id=(right,), device_id_type=pl.DeviceIdType.MESH,
        )
        ro = pltpu.make_async_remote_copy(
            src_ref=srco, dst_ref=commo.at[h],
            send_sem=sendo.at[h], recv_sem=recvo.at[h],
            device_id=(right,), device_id_type=pl.DeviceIdType.MESH,
        )
        rq.start()
        ro.start()
        rq.wait()
        ro.wait()
        g = lax.rem(my + N_DEV - 1 - h, N_DEV)
        stage_group(g, commq.at[h], commo.at[h])
        compute_group()


def kernel(x, Wq, K_ext, V_ext, Wo):
    my = lax.axis_index("i")
    xb = x[0]
    wq3 = jnp.swapaxes(Wq.reshape(D_MODEL, H_PER, DH), 0, 1)
    wo3 = Wo.reshape(H_PER, DH, D_MODEL)
    kb = jnp.swapaxes(lax.dynamic_index_in_dim(K_ext, my, 0, keepdims=False), 0, 1)
    vb = jnp.swapaxes(lax.dynamic_index_in_dim(V_ext, my, 0, keepdims=False), 0, 1)

    out = pl.pallas_call(
        _body,
        out_shape=(
            jax.ShapeDtypeStruct((SQ, D_MODEL), jnp.float32),
            jax.ShapeDtypeStruct((N_DEV - 1, H_PER, SQ, DH), jnp.float32),
            jax.ShapeDtypeStruct((N_DEV - 1, H_PER, DH, D_MODEL), jnp.float32),
        ),
        in_specs=[
            pl.BlockSpec(memory_space=pltpu.VMEM),
            pl.BlockSpec(memory_space=pl.ANY),
            pl.BlockSpec(memory_space=pl.ANY),
            pl.BlockSpec(memory_space=pl.ANY),
            pl.BlockSpec(memory_space=pl.ANY),
        ],
        out_specs=(
            pl.BlockSpec(memory_space=pltpu.VMEM),
            pl.BlockSpec(memory_space=pl.ANY),
            pl.BlockSpec(memory_space=pl.ANY),
        ),
        scratch_shapes=[
            pltpu.VMEM((H_PER, D_MODEL, DH), jnp.float32),
            pltpu.VMEM((H_PER, DH, D_MODEL), jnp.float32),
            pltpu.VMEM((H_PER, SKV, DH), jnp.float32),
            pltpu.VMEM((H_PER, SKV, DH), jnp.float32),
            pltpu.VMEM((SQ, SKV), jnp.float32),
            pltpu.SemaphoreType.DMA((N_DEV - 1,)),
            pltpu.SemaphoreType.DMA((N_DEV - 1,)),
            pltpu.SemaphoreType.DMA((N_DEV - 1,)),
            pltpu.SemaphoreType.DMA((N_DEV - 1,)),
            pltpu.SemaphoreType.DMA((4,)),
        ],
        compiler_params=pltpu.CompilerParams(collective_id=0),
    )(xb, wq3, wo3, kb, vb)
    return out[0][None]


# device time: 212383 ns/iter; 2.4452x vs baseline; 2.4452x over previous
import jax
import jax.numpy as jnp
from jax import lax
from jax.experimental import pallas as pl
from jax.experimental.pallas import tpu as pltpu

N_DEV = 4
SQ = 1024
SKV = 1024
D_MODEL = 1024
H_PER = 8
DH = 128
SCALE = 0.08838834764831843


def _body(x_ref, wq_ref, wo_ref, k_ref, v_ref, out_ref,
          commq, commo, bias_ref, sendq, recvq, sendo, recvo):
    my = lax.axis_index("i")
    right = lax.rem(my + 1, N_DEV)
    left = lax.rem(my + N_DEV - 1, N_DEV)
    diag = lax.rem(my + 2, N_DEV)

    barrier_sem = pltpu.get_barrier_semaphore()
    for nbr in (left, right, diag):
        pl.semaphore_signal(
            barrier_sem, inc=1,
            device_id=(nbr,), device_id_type=pl.DeviceIdType.MESH,
        )
    pl.semaphore_wait(barrier_sem, 3)

    qb = lax.broadcasted_iota(jnp.int32, (SQ, SKV), 0) // 64
    kb = lax.broadcasted_iota(jnp.int32, (SQ, SKV), 1) // 64
    keep = (qb == kb) | (kb == 0) | (lax.rem(qb + kb, 3) == 0)
    bias_ref[...] = jnp.where(keep, jnp.float32(0.0), jnp.float32(-1e9))

    def mk(src, comm, slot, ssems, rsems, dev):
        return pltpu.make_async_remote_copy(
            src_ref=src, dst_ref=comm.at[slot],
            send_sem=ssems.at[slot], recv_sem=rsems.at[slot],
            device_id=(dev,), device_id_type=pl.DeviceIdType.MESH,
        )

    rq = [mk(wq_ref, commq, 0, sendq, recvq, right),
          mk(wq_ref, commq, 1, sendq, recvq, left),
          mk(wq_ref, commq, 2, sendq, recvq, diag)]
    ro = [mk(wo_ref, commo, 0, sendo, recvo, right),
          mk(wo_ref, commo, 1, sendo, recvo, left),
          mk(wo_ref, commo, 2, sendo, recvo, diag)]

    for i in (0, 1):
        rq[i].start()
        ro[i].start()

    def compute_group(g, wq_at, wo_at):
        def head_body(h, carry):
            gh = g * H_PER + h
            qh = jnp.dot(x_ref[...], wq_at(h),
                         preferred_element_type=jnp.float32
                         ).astype(jnp.bfloat16)
            s = lax.dot_general(
                qh, k_ref[gh],
                (((1,), (1,)), ((), ())),
                preferred_element_type=jnp.float32,
            ) * SCALE + bias_ref[...]
            w = jnp.exp(s - jnp.max(s, axis=1, keepdims=True))
            w = (w / jnp.sum(w, axis=1, keepdims=True)).astype(jnp.bfloat16)
            ctx = jnp.dot(w, v_ref[gh],
                          preferred_element_type=jnp.float32
                          ).astype(jnp.bfloat16)
            out_ref[...] = out_ref[...] + jnp.dot(
                ctx, wo_at(h), preferred_element_type=jnp.float32)
            return carry
        lax.fori_loop(0, H_PER, head_body, 0)

    out_ref[...] = jnp.zeros((SQ, D_MODEL), jnp.float32)
    compute_group(my, lambda h: wq_ref[h], lambda h: wo_ref[h])

    for i in (0, 1):
        rq[i].wait_send()
        ro[i].wait_send()
    rq[2].start()
    ro[2].start()

    for slot, off in ((0, N_DEV - 1), (1, 1), (2, 2)):
        rq[slot].wait_recv()
        ro[slot].wait_recv()
        g = lax.rem(my + off, N_DEV)
        compute_group(g,
                      lambda h: commq[slot, h],
                      lambda h: commo[slot, h])

    rq[2].wait_send()
    ro[2].wait_send()


def kernel(x, Wq, K_ext, V_ext, Wo):
    my = lax.axis_index("i")
    bf = jnp.bfloat16
    xb = x[0].astype(bf)
    wq3 = jnp.swapaxes(Wq.reshape(D_MODEL, H_PER, DH), 0, 1).astype(bf)
    wo3 = Wo.reshape(H_PER, DH, D_MODEL).astype(bf)
    kb = jnp.swapaxes(
        lax.dynamic_index_in_dim(K_ext, my, 0, keepdims=False), 0, 1).astype(bf)
    vb = jnp.swapaxes(
        lax.dynamic_index_in_dim(V_ext, my, 0, keepdims=False), 0, 1).astype(bf)

    out = pl.pallas_call(
        _body,
        out_shape=jax.ShapeDtypeStruct((SQ, D_MODEL), jnp.float32),
        in_specs=[
            pl.BlockSpec(memory_space=pltpu.VMEM),
            pl.BlockSpec(memory_space=pltpu.VMEM),
            pl.BlockSpec(memory_space=pltpu.VMEM),
            pl.BlockSpec(memory_space=pltpu.VMEM),
            pl.BlockSpec(memory_space=pltpu.VMEM),
        ],
        out_specs=pl.BlockSpec(memory_space=pltpu.VMEM),
        scratch_shapes=[
            pltpu.VMEM((N_DEV - 1, H_PER, D_MODEL, DH), bf),
            pltpu.VMEM((N_DEV - 1, H_PER, DH, D_MODEL), bf),
            pltpu.VMEM((SQ, SKV), jnp.float32),
            pltpu.SemaphoreType.DMA((N_DEV - 1,)),
            pltpu.SemaphoreType.DMA((N_DEV - 1,)),
            pltpu.SemaphoreType.DMA((N_DEV - 1,)),
            pltpu.SemaphoreType.DMA((N_DEV - 1,)),
        ],
        compiler_params=pltpu.CompilerParams(collective_id=0),
    )(xb, wq3, wo3, kb, vb)
    return out[None]


# device time: 175630 ns/iter; 2.9569x vs baseline; 1.2093x over previous
import jax
import jax.numpy as jnp
from jax import lax
from jax.experimental import pallas as pl
from jax.experimental.pallas import tpu as pltpu

N_DEV = 4
SQ = 1024
SKV = 1024
D_MODEL = 1024
H_PER = 8
DH = 128
SCALE = 0.08838834764831843


def _body(x_ref, wq_ref, wo_ref, k_ref, v_ref, out_ref,
          commq, commo, bias_ref, sendq, recvq, sendo, recvo):
    my = lax.axis_index("i")
    right = lax.rem(my + 1, N_DEV)
    left = lax.rem(my + N_DEV - 1, N_DEV)
    diag = lax.rem(my + 2, N_DEV)

    barrier_sem = pltpu.get_barrier_semaphore()
    for nbr in (left, right, diag):
        pl.semaphore_signal(
            barrier_sem, inc=1,
            device_id=(nbr,), device_id_type=pl.DeviceIdType.MESH,
        )
    pl.semaphore_wait(barrier_sem, 3)

    qb = lax.broadcasted_iota(jnp.int32, (SQ, SKV), 0) // 64
    kb = lax.broadcasted_iota(jnp.int32, (SQ, SKV), 1) // 64
    keep = (qb == kb) | (kb == 0) | (lax.rem(qb + kb, 3) == 0)
    bias_ref[...] = jnp.where(keep, jnp.float32(0.0), jnp.float32(-1e9))

    def mk(src, comm, slot, ssems, rsems, dev):
        return pltpu.make_async_remote_copy(
            src_ref=src, dst_ref=comm.at[slot],
            send_sem=ssems.at[slot], recv_sem=rsems.at[slot],
            device_id=(dev,), device_id_type=pl.DeviceIdType.MESH,
        )

    rq = [mk(wq_ref, commq, 0, sendq, recvq, right),
          mk(wq_ref, commq, 1, sendq, recvq, left),
          mk(wq_ref, commq, 2, sendq, recvq, diag)]
    ro = [mk(wo_ref, commo, 0, sendo, recvo, right),
          mk(wo_ref, commo, 1, sendo, recvo, left),
          mk(wo_ref, commo, 2, sendo, recvo, diag)]

    for i in (0, 1):
        rq[i].start()
        ro[i].start()

    def compute_group(g, wq_at, wo_at):
        def head_body(h, carry):
            gh = g * H_PER + h
            qh = jnp.dot(x_ref[...], wq_at(h),
                         preferred_element_type=jnp.float32
                         ).astype(jnp.bfloat16)
            s = lax.dot_general(
                qh, k_ref[gh],
                (((1,), (1,)), ((), ())),
                preferred_element_type=jnp.float32,
            )
            w = jnp.exp(s + bias_ref[...])
            wb = w.astype(jnp.bfloat16)
            denom = jnp.sum(w, axis=1, keepdims=True)
            ctx = jnp.dot(wb, v_ref[gh],
                          preferred_element_type=jnp.float32)
            ctx = (ctx / denom).astype(jnp.bfloat16)
            out_ref[...] = out_ref[...] + jnp.dot(
                ctx, wo_at(h), preferred_element_type=jnp.float32)
            return carry
        lax.fori_loop(0, H_PER, head_body, 0)

    out_ref[...] = jnp.zeros((SQ, D_MODEL), jnp.float32)
    compute_group(my, lambda h: wq_ref[h], lambda h: wo_ref[h])

    for i in (0, 1):
        rq[i].wait_send()
        ro[i].wait_send()
    rq[2].start()
    ro[2].start()

    for slot, off in ((0, N_DEV - 1), (1, 1), (2, 2)):
        rq[slot].wait_recv()
        ro[slot].wait_recv()
        g = lax.rem(my + off, N_DEV)
        compute_group(g,
                      lambda h: commq[slot, h],
                      lambda h: commo[slot, h])

    rq[2].wait_send()
    ro[2].wait_send()


def kernel(x, Wq, K_ext, V_ext, Wo):
    my = lax.axis_index("i")
    bf = jnp.bfloat16
    xb = x[0].astype(bf)
    wq3 = jnp.swapaxes(
        (Wq * SCALE).reshape(D_MODEL, H_PER, DH), 0, 1).astype(bf)
    wo3 = Wo.reshape(H_PER, DH, D_MODEL).astype(bf)
    kb = jnp.swapaxes(
        lax.dynamic_index_in_dim(K_ext, my, 0, keepdims=False), 0, 1).astype(bf)
    vb = jnp.swapaxes(
        lax.dynamic_index_in_dim(V_ext, my, 0, keepdims=False), 0, 1).astype(bf)

    out = pl.pallas_call(
        _body,
        out_shape=jax.ShapeDtypeStruct((SQ, D_MODEL), jnp.float32),
        in_specs=[
            pl.BlockSpec(memory_space=pltpu.VMEM),
            pl.BlockSpec(memory_space=pltpu.VMEM),
            pl.BlockSpec(memory_space=pltpu.VMEM),
            pl.BlockSpec(memory_space=pltpu.VMEM),
            pl.BlockSpec(memory_space=pltpu.VMEM),
        ],
        out_specs=pl.BlockSpec(memory_space=pltpu.VMEM),
        scratch_shapes=[
            pltpu.VMEM((N_DEV - 1, H_PER, D_MODEL, DH), bf),
            pltpu.VMEM((N_DEV - 1, H_PER, DH, D_MODEL), bf),
            pltpu.VMEM((SQ, SKV), jnp.float32),
            pltpu.SemaphoreType.DMA((N_DEV - 1,)),
            pltpu.SemaphoreType.DMA((N_DEV - 1,)),
            pltpu.SemaphoreType.DMA((N_DEV - 1,)),
            pltpu.SemaphoreType.DMA((N_DEV - 1,)),
        ],
        compiler_params=pltpu.CompilerParams(collective_id=0),
    )(xb, wq3, wo3, kb, vb)
    return out[None]
